# Initial kernel scaffold; baseline (speedup 1.0000x reference)
import jax
import jax.numpy as jnp
from jax.experimental import pallas as pl


def kernel(x, edge_index, batch, params, bn_stats):
    raise NotImplementedError("write your pallas kernel here")


# speedup vs baseline: 6.0742x; 6.0742x over previous
import functools

import jax
import jax.numpy as jnp
from jax import lax
from jax.experimental import pallas as pl
from jax.experimental.pallas import tpu as pltpu
from jax.experimental.pallas import tpu_sc as plsc

N = 10000
E = 320000
D = 128
G = 128
L = 3

NC = 2
NS = 16
NW = NC * NS

CHUNK = 128
N_CHUNKS = E // CHUNK
ROWS_PER_TILE = 632

_sc_mesh = plsc.VectorSubcoreMesh(
    core_axis_name="c", subcore_axis_name="s", num_cores=NC, num_subcores=NS
)


@functools.partial(
    pl.kernel,
    out_type=jax.ShapeDtypeStruct((NC, N, D), jnp.float32),
    mesh=_sc_mesh,
    scratch_types=[
        pltpu.VMEM_SHARED((N, D), jnp.float32),
        pltpu.VMEM((CHUNK,), jnp.int32),
        pltpu.VMEM((CHUNK,), jnp.int32),
        pltpu.VMEM((CHUNK, D), jnp.float32),
        pltpu.SemaphoreType.DMA,
    ],
)
def _sc_agg(h_hbm, zero_hbm, src_hbm, dst_hbm, out_hbm, acc, src_v, dst_v,
            rows_v, sem):
    c = lax.axis_index("c")
    s = lax.axis_index("s")
    wid = s * NC + c

    r0 = pl.multiple_of(jnp.minimum(s * ROWS_PER_TILE, N - ROWS_PER_TILE), 8)

    @pl.when(c == 0)
    def _():
        pltpu.sync_copy(h_hbm.at[pl.ds(r0, ROWS_PER_TILE)],
                        acc.at[pl.ds(r0, ROWS_PER_TILE)])

    @pl.when(c != 0)
    def _():
        pltpu.sync_copy(zero_hbm.at[pl.ds(r0, ROWS_PER_TILE)],
                        acc.at[pl.ds(r0, ROWS_PER_TILE)])

    plsc.subcore_barrier()

    q, r = divmod(N_CHUNKS, NW)
    n_chunks = q + jnp.where(wid < r, 1, 0)
    start = wid * q + jnp.minimum(wid, r)

    def body(i, carry):
        base = pl.multiple_of((start + i) * CHUNK, CHUNK)
        pltpu.sync_copy(src_hbm.at[pl.ds(base, CHUNK)], src_v)
        pltpu.sync_copy(dst_hbm.at[pl.ds(base, CHUNK)], dst_v)
        pltpu.async_copy(h_hbm.at[src_v], rows_v, sem).wait()
        pltpu.sync_copy(rows_v, acc.at[dst_v], add=True)
        return carry

    lax.fori_loop(0, n_chunks, body, 0)

    plsc.subcore_barrier()
    pltpu.sync_copy(acc.at[pl.ds(r0, ROWS_PER_TILE)],
                    out_hbm.at[c, pl.ds(r0, ROWS_PER_TILE)])


BLK = 1000


def _tc_mlp_body(a0_ref, a1_ref, w1_ref, b1_ref, w2_ref, b2_ref, sc_ref,
                 sh_ref, out_ref):
    h2 = a0_ref[...] + a1_ref[...]
    t = jnp.dot(h2, w1_ref[...], preferred_element_type=jnp.float32)
    t = jnp.maximum(t + b1_ref[...], 0.0)
    t = jnp.dot(t, w2_ref[...], preferred_element_type=jnp.float32)
    t = (t + b2_ref[...]) * sc_ref[...] + sh_ref[...]
    out_ref[...] = jnp.maximum(t, 0.0)


_row_spec = pl.BlockSpec((BLK, D), lambda i: (i, 0))
_full_spec = pl.BlockSpec((D, D), lambda i: (0, 0))
_vec_spec = pl.BlockSpec((1, D), lambda i: (0, 0))

_tc_mlp = pl.pallas_call(
    _tc_mlp_body,
    grid=(N // BLK,),
    in_specs=[_row_spec, _row_spec, _full_spec, _vec_spec, _full_spec,
              _vec_spec, _vec_spec, _vec_spec],
    out_specs=_row_spec,
    out_shape=jax.ShapeDtypeStruct((N, D), jnp.float32),
)


def _tc_pool_body(b_ref, h_ref, out_ref):
    @pl.when(pl.program_id(0) == 0)
    def _():
        out_ref[...] = jnp.zeros_like(out_ref)

    seg = lax.broadcasted_iota(jnp.int32, (G, BLK), 0)
    onehot = (seg == b_ref[0]).astype(jnp.float32)
    out_ref[...] += jnp.dot(onehot, h_ref[...],
                            preferred_element_type=jnp.float32)


_tc_pool = pl.pallas_call(
    _tc_pool_body,
    grid=(N // BLK,),
    in_specs=[pl.BlockSpec((1, 1, BLK), lambda i: (i, 0, 0)), _row_spec],
    out_specs=pl.BlockSpec((G, D), lambda i: (0, 0)),
    out_shape=jax.ShapeDtypeStruct((G, D), jnp.float32),
)


def kernel(x, edge_index, batch, params, bn_stats):
    src = edge_index[0]
    dst = edge_index[1]
    zeros = jnp.zeros((N, D), jnp.float32)
    batch3 = batch.reshape(N // BLK, 1, BLK)

    h = x
    for l in range(L):
        agg = _sc_agg(h, zeros, src, dst)
        scale = (params[f"g_{l}"] /
                 jnp.sqrt(bn_stats[f"rv_{l}"] + 1e-5)).reshape(1, D)
        shift = (bn_stats[f"rm_{l}"] * (-scale[0]) +
                 params[f"be_{l}"]).reshape(1, D)
        h = _tc_mlp(agg[0], agg[1], params[f"W1_{l}"],
                    params[f"b1_{l}"].reshape(1, D), params[f"W2_{l}"],
                    params[f"b2_{l}"].reshape(1, D), scale, shift)

    x_pool = _tc_pool(batch3, h)
    return (h, x_pool)
